# baseline (speedup 1.0000x reference)
import functools

import jax
import jax.numpy as jnp
from jax import lax
from jax.experimental import pallas as pl
from jax.experimental.pallas import tpu as pltpu
from jax.experimental.pallas import tpu_sc as plsc

BATCH = 16384
DIM = 64
W = 2 * DIM
PAIRS = 131072
BITS = 18
SHIFT = 32 - BITS
PAIR_SHIFT = SHIFT + 1
HASH_A_USER = 2654435761
HASH_A_ITEM = 2246822519

NC = 2
NS = 16
NW = NC * NS
B_PER_W = BATCH // NW
N_CHUNK = 8
CHUNK = B_PER_W // N_CHUNK
NBUF = 2
L = 16


def _sc_gather_body(user_hbm, item_hbm, utab_hbm, itab_hbm,
                    uout_hbm, iout_hbm,
                    raw_u, raw_i, uidx, iidx, urows, irows,
                    usem, isem):
    wid = lax.axis_index("s") * NC + lax.axis_index("c")
    base = wid * B_PER_W

    pltpu.sync_copy(user_hbm.at[pl.ds(base, B_PER_W)], raw_u)
    pltpu.sync_copy(item_hbm.at[pl.ds(base, B_PER_W)], raw_i)

    au = jnp.uint32(HASH_A_USER)
    ai = jnp.uint32(HASH_A_ITEM)
    sh = jnp.uint32(PAIR_SHIFT)
    for k in range(B_PER_W // L):
        r = k // (CHUNK // L)
        c = (k % (CHUNK // L)) * L
        u = raw_u[pl.ds(k * L, L)].astype(jnp.uint32)
        i = raw_i[pl.ds(k * L, L)].astype(jnp.uint32)
        uidx[r, pl.ds(c, L)] = ((u * au) >> sh).astype(jnp.int32)
        iidx[r, pl.ds(c, L)] = ((i * ai) >> sh).astype(jnp.int32)

    u_h = {}
    i_h = {}
    for j in range(N_CHUNK + NBUF):
        if j >= NBUF:
            k = j - NBUF
            u_h[k].wait()
            pltpu.sync_copy(urows.at[k % NBUF],
                            uout_hbm.at[pl.ds(base + k * CHUNK, CHUNK)])
            i_h[k].wait()
            pltpu.sync_copy(irows.at[k % NBUF],
                            iout_hbm.at[pl.ds(base + k * CHUNK, CHUNK)])
        if j < N_CHUNK:
            u_h[j] = pltpu.async_copy(utab_hbm.at[uidx.at[j]],
                                      urows.at[j % NBUF], usem.at[j % NBUF])
            i_h[j] = pltpu.async_copy(itab_hbm.at[iidx.at[j]],
                                      irows.at[j % NBUF], isem.at[j % NBUF])


_sc_gather = functools.partial(
    pl.kernel,
    out_type=(
        jax.ShapeDtypeStruct((BATCH, W), jnp.float32),
        jax.ShapeDtypeStruct((BATCH, W), jnp.float32),
    ),
    mesh=plsc.VectorSubcoreMesh(core_axis_name="c", subcore_axis_name="s"),
    scratch_types=[
        pltpu.VMEM((B_PER_W,), jnp.int32),
        pltpu.VMEM((B_PER_W,), jnp.int32),
        pltpu.VMEM((N_CHUNK, CHUNK), jnp.int32),
        pltpu.VMEM((N_CHUNK, CHUNK), jnp.int32),
        pltpu.VMEM((NBUF, CHUNK, W), jnp.float32),
        pltpu.VMEM((NBUF, CHUNK, W), jnp.float32),
        pltpu.SemaphoreType.DMA((NBUF,)),
        pltpu.SemaphoreType.DMA((NBUF,)),
    ],
    compiler_params=pltpu.CompilerParams(use_tc_tiling_on_sc=True),
)(_sc_gather_body)


BLK = 2048


def _mlp_body(u_ref, v_ref, uid_ref, vid_ref, w1_ref, b1_ref, w2_ref, b2_ref,
              o_ref):
    au = jnp.uint32(HASH_A_USER)
    ai = jnp.uint32(HASH_A_ITEM)
    up = ((uid_ref[...].astype(jnp.uint32) * au) >> jnp.uint32(SHIFT)) & 1
    vp = ((vid_ref[...].astype(jnp.uint32) * ai) >> jnp.uint32(SHIFT)) & 1
    u2 = u_ref[...]
    v2 = v_ref[...]
    u = jnp.where(up == 1, u2[:, DIM:], u2[:, :DIM])
    v = jnp.where(vp == 1, v2[:, DIM:], v2[:, :DIM])
    x = u * v
    h = jnp.dot(x, w1_ref[...], preferred_element_type=jnp.float32) + b1_ref[...]
    h = jnp.maximum(h, 0.0)
    z = jnp.dot(h, w2_ref[...], preferred_element_type=jnp.float32) + b2_ref[0, 0]
    o_ref[...] = 1.0 / (1.0 + jnp.exp(-z))


def _mlp(u_emb, i_emb, user, item, W1, b1, W2, b2):
    grid = (BATCH // BLK,)
    return pl.pallas_call(
        _mlp_body,
        grid=grid,
        in_specs=[
            pl.BlockSpec((BLK, W), lambda i: (i, 0)),
            pl.BlockSpec((BLK, W), lambda i: (i, 0)),
            pl.BlockSpec((BLK, 1), lambda i: (i, 0)),
            pl.BlockSpec((BLK, 1), lambda i: (i, 0)),
            pl.BlockSpec((DIM, 20), lambda i: (0, 0)),
            pl.BlockSpec((1, 20), lambda i: (0, 0)),
            pl.BlockSpec((20, 1), lambda i: (0, 0)),
            pl.BlockSpec((1, 1), lambda i: (0, 0)),
        ],
        out_specs=pl.BlockSpec((BLK, 1), lambda i: (i, 0)),
        out_shape=jax.ShapeDtypeStruct((BATCH, 1), jnp.float32),
    )(u_emb, i_emb, user.reshape(BATCH, 1), item.reshape(BATCH, 1),
      W1, b1, W2, b2)


def kernel(user, item, user_table, item_table, W1, b1, W2, b2):
    user_table, item_table = lax.optimization_barrier((user_table, item_table))
    ut2 = user_table.reshape(PAIRS, W)
    it2 = item_table.reshape(PAIRS, W)
    u_emb, i_emb = _sc_gather(user, item, ut2, it2)
    out = _mlp(u_emb, i_emb, user, item, W1,
               b1.reshape(1, 20), W2, b2.reshape(1, 1))
    return out.reshape(-1)


# speedup vs baseline: 1.2828x; 1.2828x over previous
import functools

import jax
import jax.numpy as jnp
from jax import lax
from jax.experimental import pallas as pl
from jax.experimental.pallas import tpu as pltpu
from jax.experimental.pallas import tpu_sc as plsc

BATCH = 16384
DIM = 64
W = 2 * DIM
PAIRS = 131072
BITS = 18
SHIFT = 32 - BITS
PAIR_SHIFT = SHIFT + 1
HASH_A_USER = 2654435761
HASH_A_ITEM = 2246822519

NC = 2
NS = 16
NW = NC * NS
B_PER_W = BATCH // NW
N_CHUNK = 8
CHUNK = B_PER_W // N_CHUNK
NBUF = 2
L = 16


def _sc_gather_body(hash_a, ids_hbm, tab_hbm, out_hbm,
                    raw, idx, rows, sem):
    wid = lax.axis_index("s") * NC + lax.axis_index("c")
    base = wid * B_PER_W

    pltpu.sync_copy(ids_hbm.at[pl.ds(base, B_PER_W)], raw)

    a = jnp.uint32(hash_a)
    sh = jnp.uint32(PAIR_SHIFT)
    for k in range(B_PER_W // L):
        r = k // (CHUNK // L)
        c = (k % (CHUNK // L)) * L
        v = raw[pl.ds(k * L, L)].astype(jnp.uint32)
        idx[r, pl.ds(c, L)] = ((v * a) >> sh).astype(jnp.int32)

    h = {}
    for j in range(N_CHUNK + NBUF):
        if j >= NBUF:
            k = j - NBUF
            h[k].wait()
            pltpu.sync_copy(rows.at[k % NBUF],
                            out_hbm.at[pl.ds(base + k * CHUNK, CHUNK)])
        if j < N_CHUNK:
            h[j] = pltpu.async_copy(tab_hbm.at[idx.at[j]],
                                    rows.at[j % NBUF], sem.at[j % NBUF])


def _make_sc_gather(hash_a):
    return functools.partial(
        pl.kernel,
        out_type=jax.ShapeDtypeStruct((BATCH, W), jnp.float32),
        mesh=plsc.VectorSubcoreMesh(core_axis_name="c", subcore_axis_name="s"),
        scratch_types=[
            pltpu.VMEM((B_PER_W,), jnp.int32),
            pltpu.VMEM((N_CHUNK, CHUNK), jnp.int32),
            pltpu.VMEM((NBUF, CHUNK, W), jnp.float32),
            pltpu.SemaphoreType.DMA((NBUF,)),
        ],
        compiler_params=pltpu.CompilerParams(use_tc_tiling_on_sc=True),
    )(functools.partial(_sc_gather_body, hash_a))


_sc_gather_user = _make_sc_gather(HASH_A_USER)
_sc_gather_item = _make_sc_gather(HASH_A_ITEM)


BLK = 2048


def _mlp_body(u_ref, v_ref, uid_ref, vid_ref, w1_ref, b1_ref, w2_ref, b2_ref,
              o_ref):
    au = jnp.uint32(HASH_A_USER)
    ai = jnp.uint32(HASH_A_ITEM)
    up = ((uid_ref[...].astype(jnp.uint32) * au) >> jnp.uint32(SHIFT)) & 1
    vp = ((vid_ref[...].astype(jnp.uint32) * ai) >> jnp.uint32(SHIFT)) & 1
    u2 = u_ref[...]
    v2 = v_ref[...]
    u = jnp.where(up == 1, u2[:, DIM:], u2[:, :DIM])
    v = jnp.where(vp == 1, v2[:, DIM:], v2[:, :DIM])
    x = u * v
    h = jnp.dot(x, w1_ref[...], preferred_element_type=jnp.float32) + b1_ref[...]
    h = jnp.maximum(h, 0.0)
    z = jnp.dot(h, w2_ref[...], preferred_element_type=jnp.float32) + b2_ref[0, 0]
    o_ref[...] = 1.0 / (1.0 + jnp.exp(-z))


def _mlp(u_emb, i_emb, user, item, W1, b1, W2, b2):
    grid = (BATCH // BLK,)
    return pl.pallas_call(
        _mlp_body,
        grid=grid,
        in_specs=[
            pl.BlockSpec((BLK, W), lambda i: (i, 0)),
            pl.BlockSpec((BLK, W), lambda i: (i, 0)),
            pl.BlockSpec((BLK, 1), lambda i: (i, 0)),
            pl.BlockSpec((BLK, 1), lambda i: (i, 0)),
            pl.BlockSpec((DIM, 20), lambda i: (0, 0)),
            pl.BlockSpec((1, 20), lambda i: (0, 0)),
            pl.BlockSpec((20, 1), lambda i: (0, 0)),
            pl.BlockSpec((1, 1), lambda i: (0, 0)),
        ],
        out_specs=pl.BlockSpec((BLK, 1), lambda i: (i, 0)),
        out_shape=jax.ShapeDtypeStruct((BATCH, 1), jnp.float32),
    )(u_emb, i_emb, user.reshape(BATCH, 1), item.reshape(BATCH, 1),
      W1, b1, W2, b2)


def kernel(user, item, user_table, item_table, W1, b1, W2, b2):
    ut2 = user_table.reshape(PAIRS, W)
    it2 = item_table.reshape(PAIRS, W)
    u_emb = _sc_gather_user(user, ut2)
    i_emb = _sc_gather_item(item, it2)
    out = _mlp(u_emb, i_emb, user, item, W1,
               b1.reshape(1, 20), W2, b2.reshape(1, 1))
    return out.reshape(-1)
